# Initial kernel scaffold; baseline (speedup 1.0000x reference)
import jax
import jax.numpy as jnp
from jax.experimental import pallas as pl


def kernel(class_logits, box_regression, angle_pred, proposals):
    raise NotImplementedError("write your pallas kernel here")


# speedup vs baseline: 20.0273x; 20.0273x over previous
import functools

import jax
import jax.numpy as jnp
import numpy as np
from jax.experimental import pallas as pl
from jax.experimental.pallas import tpu as pltpu

_B = 2
_N = 5000
_C = 81
_NC = _C - 1
_DET = 100
_SCORE_THRESH = 0.05
_NMS_THRESH = 0.5
_IMG_H = 800.0
_IMG_W = 800.0
_WX, _WY, _WW, _WH = 10.0, 10.0, 5.0, 5.0
_BBOX_XFORM_CLIP = float(np.log(1000.0 / 16.0))

_TN = 1000
_SUB, _LANE = 8, 640
_NPAD = _SUB * _LANE
_BIG = 2**31 - 1
_NEG = -jnp.inf


def _dense_kernel(log_ref, rdx_ref, rdy_ref, rdw_ref, rdh_ref, pr_ref,
                  scm_ref, x0_ref, y0_ref, x1_ref, y1_ref):
    l = log_ref[0]
    m = jnp.max(l, axis=1, keepdims=True)
    e = jnp.exp(l - m)
    s = jnp.sum(e, axis=1, keepdims=True)
    p = e / s
    sc = p[:, 1:]

    pr = pr_ref[0]
    w = pr[:, 2:3] - pr[:, 0:1]
    h = pr[:, 3:4] - pr[:, 1:2]
    cx = pr[:, 0:1] + 0.5 * w
    cy = pr[:, 1:2] + 0.5 * h

    dx = rdx_ref[0][:, 1:] / _WX
    dy = rdy_ref[0][:, 1:] / _WY
    dw = jnp.minimum(rdw_ref[0][:, 1:] / _WW, _BBOX_XFORM_CLIP)
    dh = jnp.minimum(rdh_ref[0][:, 1:] / _WH, _BBOX_XFORM_CLIP)

    pcx = dx * w + cx
    pcy = dy * h + cy
    pw = jnp.exp(dw) * w
    ph = jnp.exp(dh) * h

    x0 = jnp.clip(pcx - 0.5 * pw, 0.0, _IMG_W)
    y0 = jnp.clip(pcy - 0.5 * ph, 0.0, _IMG_H)
    x1 = jnp.clip(pcx + 0.5 * pw, 0.0, _IMG_W)
    y1 = jnp.clip(pcy + 0.5 * ph, 0.0, _IMG_H)

    ws = x1 - x0
    hs = y1 - y0
    valid = (sc > _SCORE_THRESH) & (ws >= 1e-2) & (hs >= 1e-2)
    scm_ref[0] = jnp.where(valid, sc, _NEG)
    x0_ref[0] = x0
    y0_ref[0] = y0
    x1_ref[0] = x1
    y1_ref[0] = y1


def _nms_kernel(scm_ref, x0_ref, y0_ref, x1_ref, y1_ref, ang_ref,
                ob_ref, osc_ref, ol_ref, oa_ref,
                s_scr, m_scr, i_scr):
    n_iota = (jax.lax.broadcasted_iota(jnp.int32, (_SUB, _LANE), 0) * _LANE
              + jax.lax.broadcasted_iota(jnp.int32, (_SUB, _LANE), 1))
    lane128 = jax.lax.broadcasted_iota(jnp.int32, (1, 128), 1)
    lane100 = jax.lax.broadcasted_iota(jnp.int32, (1, _DET), 1)

    m_scr[...] = jnp.full((8, 128), _NEG, jnp.float32)
    i_scr[...] = jnp.full((8, 128), _BIG, jnp.int32)

    def init_c(c, maxc):
        slab = scm_ref[0, c]
        s_scr[c] = slab
        mx = jnp.max(slab)
        nb = jnp.min(jnp.where(slab == mx, n_iota, _BIG))
        mrow = m_scr[0:1, :]
        m_scr[0:1, :] = jnp.where(lane128 == c, mx, mrow)
        irow = i_scr[0:1, :]
        i_scr[0:1, :] = jnp.where(lane128 == c, nb * _NC + c, irow)
        mc = jnp.maximum(
            jnp.maximum(jnp.max(x0_ref[0, c]), jnp.max(y0_ref[0, c])),
            jnp.maximum(jnp.max(x1_ref[0, c]), jnp.max(y1_ref[0, c])))
        return jnp.maximum(maxc, mc)

    maxc = jax.lax.fori_loop(0, _NC, init_c, jnp.float32(0.0))

    def step(t, carry):
        mrow = m_scr[0:1, :]
        gm = jnp.max(mrow)

        @pl.when(gm > _NEG)
        def _selected():
            irow = i_scr[0:1, :]
            gidx = jnp.min(jnp.where(mrow == gm, irow, _BIG))
            c = gidx % _NC
            n = gidx // _NC
            mask = n_iota == n

            slabx0 = x0_ref[0, c]
            slaby0 = y0_ref[0, c]
            slabx1 = x1_ref[0, c]
            slaby1 = y1_ref[0, c]
            sx0 = jnp.max(jnp.where(mask, slabx0, -1.0))
            sy0 = jnp.max(jnp.where(mask, slaby0, -1.0))
            sx1 = jnp.max(jnp.where(mask, slabx1, -1.0))
            sy1 = jnp.max(jnp.where(mask, slaby1, -1.0))

            off = (c + 1).astype(jnp.float32) * (maxc + 1.0)
            sx0o, sy0o = sx0 + off, sy0 + off
            sx1o, sy1o = sx1 + off, sy1 + off
            x0o, y0o = slabx0 + off, slaby0 + off
            x1o, y1o = slabx1 + off, slaby1 + off

            ltx = jnp.maximum(sx0o, x0o)
            lty = jnp.maximum(sy0o, y0o)
            rbx = jnp.minimum(sx1o, x1o)
            rby = jnp.minimum(sy1o, y1o)
            iw = jnp.maximum(rbx - ltx, 0.0)
            ih = jnp.maximum(rby - lty, 0.0)
            inter = iw * ih
            a1 = (sx1o - sx0o) * (sy1o - sy0o)
            a2 = (x1o - x0o) * (y1o - y0o)
            iou = inter / (a1 + a2 - inter + 1e-9)

            s_slab = s_scr[c]
            s_new = jnp.where(iou > _NMS_THRESH, _NEG, s_slab)
            s_scr[c] = s_new

            m2 = jnp.max(s_new)
            nb2 = jnp.min(jnp.where(s_new == m2, n_iota, _BIG))
            m_scr[0:1, :] = jnp.where(lane128 == c, m2, mrow)
            i_scr[0:1, :] = jnp.where(lane128 == c, nb2 * _NC + c, irow)

            ang = jnp.max(jnp.where(mask, ang_ref[0], _NEG))
            sel = lane100 == t
            osc_ref[0, 0:1, :] = jnp.where(sel, gm, osc_ref[0, 0:1, :])
            ol_ref[0, 0:1, :] = jnp.where(sel, c + 1, ol_ref[0, 0:1, :])
            oa_ref[0, 0:1, :] = jnp.where(sel, ang, oa_ref[0, 0:1, :])
            ob_ref[0, 0:1, :] = jnp.where(sel, sx0, ob_ref[0, 0:1, :])
            ob_ref[0, 1:2, :] = jnp.where(sel, sy0, ob_ref[0, 1:2, :])
            ob_ref[0, 2:3, :] = jnp.where(sel, sx1, ob_ref[0, 2:3, :])
            ob_ref[0, 3:4, :] = jnp.where(sel, sy1, ob_ref[0, 3:4, :])

        @pl.when(jnp.logical_not(gm > _NEG))
        def _exhausted():
            s0 = jnp.sum(scm_ref[0, 0, 0:1, 0:1])
            kv0 = s0 > _NEG
            bx0 = jnp.sum(x0_ref[0, 0, 0:1, 0:1])
            by0 = jnp.sum(y0_ref[0, 0, 0:1, 0:1])
            bx1 = jnp.sum(x1_ref[0, 0, 0:1, 0:1])
            by1 = jnp.sum(y1_ref[0, 0, 0:1, 0:1])
            a0 = jnp.sum(ang_ref[0, 0:1, 0:1])
            zf = jnp.float32(0.0)
            sel = lane100 == t
            osc_ref[0, 0:1, :] = jnp.where(sel, jnp.where(kv0, s0, zf),
                                           osc_ref[0, 0:1, :])
            ol_ref[0, 0:1, :] = jnp.where(sel, jnp.where(kv0, 1, 0),
                                          ol_ref[0, 0:1, :])
            oa_ref[0, 0:1, :] = jnp.where(sel, jnp.where(kv0, a0, zf),
                                          oa_ref[0, 0:1, :])
            ob_ref[0, 0:1, :] = jnp.where(sel, jnp.where(kv0, bx0, zf),
                                          ob_ref[0, 0:1, :])
            ob_ref[0, 1:2, :] = jnp.where(sel, jnp.where(kv0, by0, zf),
                                          ob_ref[0, 1:2, :])
            ob_ref[0, 2:3, :] = jnp.where(sel, jnp.where(kv0, bx1, zf),
                                          ob_ref[0, 2:3, :])
            ob_ref[0, 3:4, :] = jnp.where(sel, jnp.where(kv0, by1, zf),
                                          ob_ref[0, 3:4, :])

        return carry

    jax.lax.fori_loop(0, _DET, step, jnp.int32(0))


def _cm_pad(a, padval):
    t = jnp.transpose(a, (0, 2, 1))
    t = jnp.pad(t, ((0, 0), (0, 0), (0, _NPAD - _N)), constant_values=padval)
    return t.reshape(_B, _NC, _SUB, _LANE)


@jax.jit
def kernel(class_logits, box_regression, angle_pred, proposals):
    r = box_regression.reshape(_B, _N, _C, 4)
    rdx = r[..., 0]
    rdy = r[..., 1]
    rdw = r[..., 2]
    rdh = r[..., 3]

    ntiles = _N // _TN
    f32 = jnp.float32
    dense = pl.pallas_call(
        _dense_kernel,
        grid=(_B, ntiles),
        in_specs=[
            pl.BlockSpec((1, _TN, _C), lambda b, t: (b, t, 0)),
            pl.BlockSpec((1, _TN, _C), lambda b, t: (b, t, 0)),
            pl.BlockSpec((1, _TN, _C), lambda b, t: (b, t, 0)),
            pl.BlockSpec((1, _TN, _C), lambda b, t: (b, t, 0)),
            pl.BlockSpec((1, _TN, _C), lambda b, t: (b, t, 0)),
            pl.BlockSpec((1, _TN, 4), lambda b, t: (b, t, 0)),
        ],
        out_specs=[
            pl.BlockSpec((1, _TN, _NC), lambda b, t: (b, t, 0))
            for _ in range(5)
        ],
        out_shape=[jax.ShapeDtypeStruct((_B, _N, _NC), f32) for _ in range(5)],
        compiler_params=pltpu.CompilerParams(
            dimension_semantics=("parallel", "arbitrary")),
    )
    scm, x0, y0, x1, y1 = dense(class_logits, rdx, rdy, rdw, rdh, proposals)

    scm_t = _cm_pad(scm, _NEG)
    x0t = _cm_pad(x0, 0.0)
    y0t = _cm_pad(y0, 0.0)
    x1t = _cm_pad(x1, 0.0)
    y1t = _cm_pad(y1, 0.0)
    ang_t = jnp.pad(jnp.transpose(angle_pred, (0, 2, 1)),
                    ((0, 0), (0, 0), (0, _NPAD - _N)))
    ang_t = ang_t.reshape(_B, _SUB, _LANE)

    slab_spec = pl.BlockSpec((1, _NC, _SUB, _LANE), lambda b: (b, 0, 0, 0))
    nms = pl.pallas_call(
        _nms_kernel,
        grid=(_B,),
        in_specs=[slab_spec, slab_spec, slab_spec, slab_spec, slab_spec,
                  pl.BlockSpec((1, _SUB, _LANE), lambda b: (b, 0, 0))],
        out_specs=[
            pl.BlockSpec((1, 4, _DET), lambda b: (b, 0, 0)),
            pl.BlockSpec((1, 1, _DET), lambda b: (b, 0, 0)),
            pl.BlockSpec((1, 1, _DET), lambda b: (b, 0, 0)),
            pl.BlockSpec((1, 1, _DET), lambda b: (b, 0, 0)),
        ],
        out_shape=[
            jax.ShapeDtypeStruct((_B, 4, _DET), f32),
            jax.ShapeDtypeStruct((_B, 1, _DET), f32),
            jax.ShapeDtypeStruct((_B, 1, _DET), jnp.int32),
            jax.ShapeDtypeStruct((_B, 1, _DET), f32),
        ],
        scratch_shapes=[
            pltpu.VMEM((_NC, _SUB, _LANE), f32),
            pltpu.VMEM((8, 128), f32),
            pltpu.VMEM((8, 128), jnp.int32),
        ],
        compiler_params=pltpu.CompilerParams(
            dimension_semantics=("parallel",)),
    )
    ob_t, osc, ol, oa = nms(scm_t, x0t, y0t, x1t, y1t, ang_t)
    return (jnp.transpose(ob_t, (0, 2, 1)), osc[:, 0, :],
            ol[:, 0, :], oa[:, 0, :])


# speedup vs baseline: 20.4361x; 1.0204x over previous
import jax
import jax.numpy as jnp
import numpy as np
from jax.experimental import pallas as pl
from jax.experimental.pallas import tpu as pltpu

_B = 2
_N = 5000
_C = 81
_NC = _C - 1
_DET = 100
_SCORE_THRESH = 0.05
_NMS_THRESH = 0.5
_IMG_H = 800.0
_IMG_W = 800.0
_WX, _WY, _WW, _WH = 10.0, 10.0, 5.0, 5.0
_BBOX_XFORM_CLIP = float(np.log(1000.0 / 16.0))

_TN = 1000
_SUB, _LANE = 8, 625
_BIG = 2**31 - 1
_NEG = -jnp.inf


def _dense_kernel(log_ref, rdx_ref, rdy_ref, rdw_ref, rdh_ref, pr_ref,
                  scm_ref, x0_ref, y0_ref, x1_ref, y1_ref):
    l = log_ref[0]
    m = jnp.max(l, axis=1, keepdims=True)
    e = jnp.exp(l - m)
    s = jnp.sum(e, axis=1, keepdims=True)
    p = e / s
    sc = p[:, 1:]

    pr = pr_ref[0]
    w = pr[:, 2:3] - pr[:, 0:1]
    h = pr[:, 3:4] - pr[:, 1:2]
    cx = pr[:, 0:1] + 0.5 * w
    cy = pr[:, 1:2] + 0.5 * h

    dx = rdx_ref[0][:, 1:] / _WX
    dy = rdy_ref[0][:, 1:] / _WY
    dw = jnp.minimum(rdw_ref[0][:, 1:] / _WW, _BBOX_XFORM_CLIP)
    dh = jnp.minimum(rdh_ref[0][:, 1:] / _WH, _BBOX_XFORM_CLIP)

    pcx = dx * w + cx
    pcy = dy * h + cy
    pw = jnp.exp(dw) * w
    ph = jnp.exp(dh) * h

    x0 = jnp.clip(pcx - 0.5 * pw, 0.0, _IMG_W)
    y0 = jnp.clip(pcy - 0.5 * ph, 0.0, _IMG_H)
    x1 = jnp.clip(pcx + 0.5 * pw, 0.0, _IMG_W)
    y1 = jnp.clip(pcy + 0.5 * ph, 0.0, _IMG_H)

    ws = x1 - x0
    hs = y1 - y0
    valid = (sc > _SCORE_THRESH) & (ws >= 1e-2) & (hs >= 1e-2)
    scm_ref[0] = jnp.where(valid, sc, _NEG)
    x0_ref[0] = x0
    y0_ref[0] = y0
    x1_ref[0] = x1
    y1_ref[0] = y1


def _nms_kernel(scm_ref, x0_ref, y0_ref, x1_ref, y1_ref, ang_ref,
                ob_ref, osc_ref, ol_ref, oa_ref,
                s_scr, m_scr, i_scr):
    n_iota = (jax.lax.broadcasted_iota(jnp.int32, (_SUB, _LANE), 0) * _LANE
              + jax.lax.broadcasted_iota(jnp.int32, (_SUB, _LANE), 1))
    lane128 = jax.lax.broadcasted_iota(jnp.int32, (1, 128), 1)
    lane100 = jax.lax.broadcasted_iota(jnp.int32, (1, _DET), 1)

    m_scr[...] = jnp.full((8, 128), _NEG, jnp.float32)
    i_scr[...] = jnp.full((8, 128), _BIG, jnp.int32)

    maxcs = []
    fallbacks = []
    for b in range(_B):
        def init_c(c, maxc, b=b):
            slab = scm_ref[b, c]
            s_scr[b * _NC + c] = slab
            mx = jnp.max(slab)
            nb = jnp.min(jnp.where(slab == mx, n_iota, _BIG))
            m_scr[b:b + 1, :] = jnp.where(lane128 == c, mx, m_scr[b:b + 1, :])
            i_scr[b:b + 1, :] = jnp.where(lane128 == c, nb * _NC + c,
                                          i_scr[b:b + 1, :])
            mc = jnp.maximum(
                jnp.maximum(jnp.max(x0_ref[b, c]), jnp.max(y0_ref[b, c])),
                jnp.maximum(jnp.max(x1_ref[b, c]), jnp.max(y1_ref[b, c])))
            return jnp.maximum(maxc, mc)

        maxcs.append(jax.lax.fori_loop(0, _NC, init_c, jnp.float32(0.0)))
        s0 = jnp.sum(scm_ref[b, 0, 0:1, 0:1])
        kv0 = s0 > _NEG
        zf = jnp.float32(0.0)
        fallbacks.append((
            jnp.where(kv0, s0, zf),
            jnp.where(kv0, 1, 0),
            jnp.where(kv0, jnp.sum(ang_ref[b, 0:1, 0:1]), zf),
            jnp.where(kv0, jnp.sum(x0_ref[b, 0, 0:1, 0:1]), zf),
            jnp.where(kv0, jnp.sum(y0_ref[b, 0, 0:1, 0:1]), zf),
            jnp.where(kv0, jnp.sum(x1_ref[b, 0, 0:1, 0:1]), zf),
            jnp.where(kv0, jnp.sum(y1_ref[b, 0, 0:1, 0:1]), zf),
        ))

    def step(t, carry):
        sel_t = lane100 == t
        for b in range(_B):
            mrow = m_scr[b:b + 1, :]
            gm = jnp.max(mrow)
            kv = gm > _NEG
            irow = i_scr[b:b + 1, :]
            gidx = jnp.min(jnp.where(mrow == gm, irow, _BIG))
            gidx = jnp.where(kv, gidx, 0)
            c = gidx % _NC
            n = gidx // _NC
            mask = n_iota == n

            slabx0 = x0_ref[b, c]
            slaby0 = y0_ref[b, c]
            slabx1 = x1_ref[b, c]
            slaby1 = y1_ref[b, c]
            sx0 = jnp.max(jnp.where(mask, slabx0, -1.0))
            sy0 = jnp.max(jnp.where(mask, slaby0, -1.0))
            sx1 = jnp.max(jnp.where(mask, slabx1, -1.0))
            sy1 = jnp.max(jnp.where(mask, slaby1, -1.0))

            off = (c + 1).astype(jnp.float32) * (maxcs[b] + 1.0)
            sx0o, sy0o = sx0 + off, sy0 + off
            sx1o, sy1o = sx1 + off, sy1 + off
            x0o, y0o = slabx0 + off, slaby0 + off
            x1o, y1o = slabx1 + off, slaby1 + off

            ltx = jnp.maximum(sx0o, x0o)
            lty = jnp.maximum(sy0o, y0o)
            rbx = jnp.minimum(sx1o, x1o)
            rby = jnp.minimum(sy1o, y1o)
            iw = jnp.maximum(rbx - ltx, 0.0)
            ih = jnp.maximum(rby - lty, 0.0)
            inter = iw * ih
            a1 = (sx1o - sx0o) * (sy1o - sy0o)
            a2 = (x1o - x0o) * (y1o - y0o)
            iou = inter / (a1 + a2 - inter + 1e-9)

            s_slab = s_scr[b * _NC + c]
            s_new = jnp.where(iou > _NMS_THRESH, _NEG, s_slab)
            s_scr[b * _NC + c] = s_new

            m2 = jnp.max(s_new)
            nb2 = jnp.min(jnp.where(s_new == m2, n_iota, _BIG))
            m_scr[b:b + 1, :] = jnp.where(lane128 == c, m2, mrow)
            i_scr[b:b + 1, :] = jnp.where(lane128 == c, nb2 * _NC + c, irow)

            ang = jnp.max(jnp.where(mask, ang_ref[b], _NEG))
            fb_s, fb_l, fb_a, fb_x0, fb_y0, fb_x1, fb_y1 = fallbacks[b]
            osc_ref[b, 0:1, :] = jnp.where(sel_t, jnp.where(kv, gm, fb_s),
                                           osc_ref[b, 0:1, :])
            ol_ref[b, 0:1, :] = jnp.where(sel_t, jnp.where(kv, c + 1, fb_l),
                                          ol_ref[b, 0:1, :])
            oa_ref[b, 0:1, :] = jnp.where(sel_t, jnp.where(kv, ang, fb_a),
                                          oa_ref[b, 0:1, :])
            ob_ref[b, 0:1, :] = jnp.where(sel_t, jnp.where(kv, sx0, fb_x0),
                                          ob_ref[b, 0:1, :])
            ob_ref[b, 1:2, :] = jnp.where(sel_t, jnp.where(kv, sy0, fb_y0),
                                          ob_ref[b, 1:2, :])
            ob_ref[b, 2:3, :] = jnp.where(sel_t, jnp.where(kv, sx1, fb_x1),
                                          ob_ref[b, 2:3, :])
            ob_ref[b, 3:4, :] = jnp.where(sel_t, jnp.where(kv, sy1, fb_y1),
                                          ob_ref[b, 3:4, :])
        return carry

    jax.lax.fori_loop(0, _DET, step, jnp.int32(0))


def _cm(a):
    return jnp.transpose(a, (0, 2, 1)).reshape(_B, _NC, _SUB, _LANE)


@jax.jit
def kernel(class_logits, box_regression, angle_pred, proposals):
    r = box_regression.reshape(_B, _N, _C, 4)
    rdx = r[..., 0]
    rdy = r[..., 1]
    rdw = r[..., 2]
    rdh = r[..., 3]

    ntiles = _N // _TN
    f32 = jnp.float32
    dense = pl.pallas_call(
        _dense_kernel,
        grid=(_B, ntiles),
        in_specs=[
            pl.BlockSpec((1, _TN, _C), lambda b, t: (b, t, 0)),
            pl.BlockSpec((1, _TN, _C), lambda b, t: (b, t, 0)),
            pl.BlockSpec((1, _TN, _C), lambda b, t: (b, t, 0)),
            pl.BlockSpec((1, _TN, _C), lambda b, t: (b, t, 0)),
            pl.BlockSpec((1, _TN, _C), lambda b, t: (b, t, 0)),
            pl.BlockSpec((1, _TN, 4), lambda b, t: (b, t, 0)),
        ],
        out_specs=[
            pl.BlockSpec((1, _TN, _NC), lambda b, t: (b, t, 0))
            for _ in range(5)
        ],
        out_shape=[jax.ShapeDtypeStruct((_B, _N, _NC), f32) for _ in range(5)],
        compiler_params=pltpu.CompilerParams(
            dimension_semantics=("parallel", "arbitrary")),
    )
    scm, x0, y0, x1, y1 = dense(class_logits, rdx, rdy, rdw, rdh, proposals)

    scm_t = _cm(scm)
    x0t = _cm(x0)
    y0t = _cm(y0)
    x1t = _cm(x1)
    y1t = _cm(y1)
    ang_t = jnp.transpose(angle_pred, (0, 2, 1)).reshape(_B, _SUB, _LANE)

    slab_spec = pl.BlockSpec((_B, _NC, _SUB, _LANE), lambda: (0, 0, 0, 0))
    nms = pl.pallas_call(
        _nms_kernel,
        grid=(),
        in_specs=[slab_spec, slab_spec, slab_spec, slab_spec, slab_spec,
                  pl.BlockSpec((_B, _SUB, _LANE), lambda: (0, 0, 0))],
        out_specs=[
            pl.BlockSpec((_B, 4, _DET), lambda: (0, 0, 0)),
            pl.BlockSpec((_B, 1, _DET), lambda: (0, 0, 0)),
            pl.BlockSpec((_B, 1, _DET), lambda: (0, 0, 0)),
            pl.BlockSpec((_B, 1, _DET), lambda: (0, 0, 0)),
        ],
        out_shape=[
            jax.ShapeDtypeStruct((_B, 4, _DET), f32),
            jax.ShapeDtypeStruct((_B, 1, _DET), f32),
            jax.ShapeDtypeStruct((_B, 1, _DET), jnp.int32),
            jax.ShapeDtypeStruct((_B, 1, _DET), f32),
        ],
        scratch_shapes=[
            pltpu.VMEM((_B * _NC, _SUB, _LANE), f32),
            pltpu.VMEM((8, 128), f32),
            pltpu.VMEM((8, 128), jnp.int32),
        ],
    )
    ob_t, osc, ol, oa = nms(scm_t, x0t, y0t, x1t, y1t, ang_t)
    return (jnp.transpose(ob_t, (0, 2, 1)), osc[:, 0, :],
            ol[:, 0, :], oa[:, 0, :])


# speedup vs baseline: 23.4666x; 1.1483x over previous
import jax
import jax.numpy as jnp
import numpy as np
from jax.experimental import pallas as pl
from jax.experimental.pallas import tpu as pltpu

_B = 2
_N = 5000
_C = 81
_NC = _C - 1
_DET = 100
_SCORE_THRESH = 0.05
_NMS_THRESH = 0.5
_IMG_H = 800.0
_IMG_W = 800.0
_WX, _WY, _WW, _WH = 10.0, 10.0, 5.0, 5.0
_BBOX_XFORM_CLIP = float(np.log(1000.0 / 16.0))

_TN = 1000
_SUB, _LANE = 8, 625
_BIG = 2**31 - 1
_NEG = -jnp.inf


def _dense_kernel(log_ref, rdx_ref, rdy_ref, rdw_ref, rdh_ref, pr_ref,
                  scm_ref, x0_ref, y0_ref, x1_ref, y1_ref):
    l = log_ref[0]
    m = jnp.max(l, axis=1, keepdims=True)
    e = jnp.exp(l - m)
    s = jnp.sum(e, axis=1, keepdims=True)
    p = e / s
    sc = p[:, 1:]

    pr = pr_ref[0]
    w = pr[:, 2:3] - pr[:, 0:1]
    h = pr[:, 3:4] - pr[:, 1:2]
    cx = pr[:, 0:1] + 0.5 * w
    cy = pr[:, 1:2] + 0.5 * h

    dx = rdx_ref[0][:, 1:] / _WX
    dy = rdy_ref[0][:, 1:] / _WY
    dw = jnp.minimum(rdw_ref[0][:, 1:] / _WW, _BBOX_XFORM_CLIP)
    dh = jnp.minimum(rdh_ref[0][:, 1:] / _WH, _BBOX_XFORM_CLIP)

    pcx = dx * w + cx
    pcy = dy * h + cy
    pw = jnp.exp(dw) * w
    ph = jnp.exp(dh) * h

    x0 = jnp.clip(pcx - 0.5 * pw, 0.0, _IMG_W)
    y0 = jnp.clip(pcy - 0.5 * ph, 0.0, _IMG_H)
    x1 = jnp.clip(pcx + 0.5 * pw, 0.0, _IMG_W)
    y1 = jnp.clip(pcy + 0.5 * ph, 0.0, _IMG_H)

    ws = x1 - x0
    hs = y1 - y0
    valid = (sc > _SCORE_THRESH) & (ws >= 1e-2) & (hs >= 1e-2)
    scm_ref[0] = jnp.where(valid, sc, _NEG)
    x0_ref[0] = x0
    y0_ref[0] = y0
    x1_ref[0] = x1
    y1_ref[0] = y1


def _nms_kernel(*refs):
    ins = [refs[6 * b:6 * b + 6] for b in range(_B)]
    outs = [refs[6 * _B + 4 * b:6 * _B + 4 * b + 4] for b in range(_B)]
    scrs = [refs[10 * _B + 8 * b:10 * _B + 8 * b + 8] for b in range(_B)]

    n2 = (jax.lax.broadcasted_iota(jnp.int32, (_SUB, _LANE), 0) * _LANE
          + jax.lax.broadcasted_iota(jnp.int32, (_SUB, _LANE), 1))
    n3 = (jax.lax.broadcasted_iota(jnp.int32, (_NC, _SUB, _LANE), 1) * _LANE
          + jax.lax.broadcasted_iota(jnp.int32, (_NC, _SUB, _LANE), 2))
    cls80 = jax.lax.broadcasted_iota(jnp.int32, (_NC, 1), 0)
    lane100 = jax.lax.broadcasted_iota(jnp.int32, (1, _DET), 1)

    maxcs = []
    fallbacks = []
    for b in range(_B):
        scm_ref, x0_ref, y0_ref, x1_ref, y1_ref, ang_ref = ins[b]
        s_scr, m_scr, i_scr, cx0_scr, cy0_scr, cx1_scr, cy1_scr, ca_scr = \
            scrs[b]

        scm = scm_ref[...]
        s_scr[...] = scm
        x0a = x0_ref[...]
        y0a = y0_ref[...]
        x1a = x1_ref[...]
        y1a = y1_ref[...]
        anga = ang_ref[...]

        mx = jnp.max(scm, axis=(1, 2), keepdims=True)
        nb = jnp.min(jnp.where(scm == mx, n3, _BIG),
                     axis=(1, 2), keepdims=True)
        best = n3 == nb
        cx0 = jnp.max(jnp.where(best, x0a, -1.0), axis=(1, 2), keepdims=True)
        cy0 = jnp.max(jnp.where(best, y0a, -1.0), axis=(1, 2), keepdims=True)
        cx1 = jnp.max(jnp.where(best, x1a, -1.0), axis=(1, 2), keepdims=True)
        cy1 = jnp.max(jnp.where(best, y1a, -1.0), axis=(1, 2), keepdims=True)
        can = jnp.max(jnp.where(best, anga[None], _NEG),
                      axis=(1, 2), keepdims=True)
        m_scr[...] = mx[:, 0, :]
        i_scr[...] = nb[:, 0, :]
        cx0_scr[...] = cx0[:, 0, :]
        cy0_scr[...] = cy0[:, 0, :]
        cx1_scr[...] = cx1[:, 0, :]
        cy1_scr[...] = cy1[:, 0, :]
        ca_scr[...] = can[:, 0, :]

        maxcs.append(jnp.max(jnp.maximum(jnp.maximum(x0a, y0a),
                                         jnp.maximum(x1a, y1a))))

        s0 = jnp.sum(scm_ref[0, 0:1, 0:1])
        kv0 = s0 > _NEG
        zf = jnp.float32(0.0)
        fallbacks.append((
            jnp.where(kv0, s0, zf),
            jnp.where(kv0, 1, 0),
            jnp.where(kv0, jnp.sum(ang_ref[0:1, 0:1]), zf),
            jnp.where(kv0, jnp.sum(x0_ref[0, 0:1, 0:1]), zf),
            jnp.where(kv0, jnp.sum(y0_ref[0, 0:1, 0:1]), zf),
            jnp.where(kv0, jnp.sum(x1_ref[0, 0:1, 0:1]), zf),
            jnp.where(kv0, jnp.sum(y1_ref[0, 0:1, 0:1]), zf),
        ))

    def step(t, carry):
        sel_t = lane100 == t
        for b in range(_B):
            scm_ref, x0_ref, y0_ref, x1_ref, y1_ref, ang_ref = ins[b]
            ob_ref, osc_ref, ol_ref, oa_ref = outs[b]
            s_scr, m_scr, i_scr, cx0_scr, cy0_scr, cx1_scr, cy1_scr, \
                ca_scr = scrs[b]

            m = m_scr[...]
            gm11 = jnp.max(m, axis=0, keepdims=True)
            orig = i_scr[...] * _NC + cls80
            kv11 = gm11 > _NEG
            cand = jnp.where((m == gm11) & kv11, orig,
                             jnp.where(kv11, _BIG, 0))
            gidx = jnp.sum(jnp.min(cand, axis=0, keepdims=True))
            gm_s = jnp.sum(gm11)
            kv = gm_s > _NEG
            c = gidx % _NC
            n = gidx // _NC

            sx0 = jnp.sum(cx0_scr[pl.ds(c, 1), :])
            sy0 = jnp.sum(cy0_scr[pl.ds(c, 1), :])
            sx1 = jnp.sum(cx1_scr[pl.ds(c, 1), :])
            sy1 = jnp.sum(cy1_scr[pl.ds(c, 1), :])
            sang = jnp.sum(ca_scr[pl.ds(c, 1), :])

            fb_s, fb_l, fb_a, fb_x0, fb_y0, fb_x1, fb_y1 = fallbacks[b]
            osc_ref[0:1, :] = jnp.where(sel_t, jnp.where(kv, gm_s, fb_s),
                                        osc_ref[0:1, :])
            ol_ref[0:1, :] = jnp.where(sel_t, jnp.where(kv, c + 1, fb_l),
                                       ol_ref[0:1, :])
            oa_ref[0:1, :] = jnp.where(sel_t, jnp.where(kv, sang, fb_a),
                                       oa_ref[0:1, :])
            ob_ref[0:1, :] = jnp.where(sel_t, jnp.where(kv, sx0, fb_x0),
                                       ob_ref[0:1, :])
            ob_ref[1:2, :] = jnp.where(sel_t, jnp.where(kv, sy0, fb_y0),
                                       ob_ref[1:2, :])
            ob_ref[2:3, :] = jnp.where(sel_t, jnp.where(kv, sx1, fb_x1),
                                       ob_ref[2:3, :])
            ob_ref[3:4, :] = jnp.where(sel_t, jnp.where(kv, sy1, fb_y1),
                                       ob_ref[3:4, :])

            slabx0 = x0_ref[c]
            slaby0 = y0_ref[c]
            slabx1 = x1_ref[c]
            slaby1 = y1_ref[c]
            off = (c + 1).astype(jnp.float32) * (maxcs[b] + 1.0)
            sx0o, sy0o = sx0 + off, sy0 + off
            sx1o, sy1o = sx1 + off, sy1 + off
            x0o, y0o = slabx0 + off, slaby0 + off
            x1o, y1o = slabx1 + off, slaby1 + off

            ltx = jnp.maximum(sx0o, x0o)
            lty = jnp.maximum(sy0o, y0o)
            rbx = jnp.minimum(sx1o, x1o)
            rby = jnp.minimum(sy1o, y1o)
            iw = jnp.maximum(rbx - ltx, 0.0)
            ih = jnp.maximum(rby - lty, 0.0)
            inter = iw * ih
            a1 = (sx1o - sx0o) * (sy1o - sy0o)
            a2 = (x1o - x0o) * (y1o - y0o)
            iou = inter / (a1 + a2 - inter + 1e-9)

            s_slab = s_scr[c]
            s_new = jnp.where(iou > _NMS_THRESH, _NEG, s_slab)
            s_scr[c] = s_new

            m2 = jnp.max(s_new, axis=(0, 1), keepdims=True)
            eq = s_new == m2
            nb2 = jnp.min(jnp.where(eq, n2, _BIG),
                          axis=(0, 1), keepdims=True)
            neg1 = jnp.float32(-1.0)
            nx0 = jnp.max(jnp.where(eq, slabx0, neg1), axis=(0, 1),
                          keepdims=True)
            ny0 = jnp.max(jnp.where(eq, slaby0, neg1), axis=(0, 1),
                          keepdims=True)
            nx1 = jnp.max(jnp.where(eq, slabx1, neg1), axis=(0, 1),
                          keepdims=True)
            ny1 = jnp.max(jnp.where(eq, slaby1, neg1), axis=(0, 1),
                          keepdims=True)
            nan_ = jnp.max(jnp.where(eq, ang_ref[...], _NEG), axis=(0, 1),
                           keepdims=True)
            cnt = jnp.sum(jnp.where(eq, 1, 0), axis=(0, 1), keepdims=True)

            m_scr[pl.ds(c, 1), :] = m2
            i_scr[pl.ds(c, 1), :] = nb2
            cx0_scr[pl.ds(c, 1), :] = nx0
            cy0_scr[pl.ds(c, 1), :] = ny0
            cx1_scr[pl.ds(c, 1), :] = nx1
            cy1_scr[pl.ds(c, 1), :] = ny1
            ca_scr[pl.ds(c, 1), :] = nan_

            @pl.when(jnp.sum(cnt) > 1)
            def _tie_fixup(c=c, s_new=s_new, nb2=nb2, slabx0=slabx0,
                           slaby0=slaby0, slabx1=slabx1, slaby1=slaby1,
                           ang_ref=ang_ref, cx0_scr=cx0_scr, cy0_scr=cy0_scr,
                           cx1_scr=cx1_scr, cy1_scr=cy1_scr, ca_scr=ca_scr):
                best2 = n2 == nb2
                cx0_scr[pl.ds(c, 1), :] = jnp.max(
                    jnp.where(best2, slabx0, neg1), axis=(0, 1), keepdims=True)
                cy0_scr[pl.ds(c, 1), :] = jnp.max(
                    jnp.where(best2, slaby0, neg1), axis=(0, 1), keepdims=True)
                cx1_scr[pl.ds(c, 1), :] = jnp.max(
                    jnp.where(best2, slabx1, neg1), axis=(0, 1), keepdims=True)
                cy1_scr[pl.ds(c, 1), :] = jnp.max(
                    jnp.where(best2, slaby1, neg1), axis=(0, 1), keepdims=True)
                ca_scr[pl.ds(c, 1), :] = jnp.max(
                    jnp.where(best2, ang_ref[...], _NEG), axis=(0, 1),
                    keepdims=True)
        return carry

    jax.lax.fori_loop(0, _DET, step, jnp.int32(0))


def _cm(a):
    return jnp.transpose(a, (0, 2, 1)).reshape(_B, _NC, _SUB, _LANE)


@jax.jit
def kernel(class_logits, box_regression, angle_pred, proposals):
    r = box_regression.reshape(_B, _N, _C, 4)
    rdx = r[..., 0]
    rdy = r[..., 1]
    rdw = r[..., 2]
    rdh = r[..., 3]

    ntiles = _N // _TN
    f32 = jnp.float32
    dense = pl.pallas_call(
        _dense_kernel,
        grid=(_B, ntiles),
        in_specs=[
            pl.BlockSpec((1, _TN, _C), lambda b, t: (b, t, 0)),
            pl.BlockSpec((1, _TN, _C), lambda b, t: (b, t, 0)),
            pl.BlockSpec((1, _TN, _C), lambda b, t: (b, t, 0)),
            pl.BlockSpec((1, _TN, _C), lambda b, t: (b, t, 0)),
            pl.BlockSpec((1, _TN, _C), lambda b, t: (b, t, 0)),
            pl.BlockSpec((1, _TN, 4), lambda b, t: (b, t, 0)),
        ],
        out_specs=[
            pl.BlockSpec((1, _TN, _NC), lambda b, t: (b, t, 0))
            for _ in range(5)
        ],
        out_shape=[jax.ShapeDtypeStruct((_B, _N, _NC), f32) for _ in range(5)],
        compiler_params=pltpu.CompilerParams(
            dimension_semantics=("parallel", "arbitrary")),
    )
    scm, x0, y0, x1, y1 = dense(class_logits, rdx, rdy, rdw, rdh, proposals)

    scm_t = _cm(scm)
    x0t = _cm(x0)
    y0t = _cm(y0)
    x1t = _cm(x1)
    y1t = _cm(y1)
    ang_t = jnp.transpose(angle_pred, (0, 2, 1)).reshape(_B, _SUB, _LANE)

    slab_spec = pl.BlockSpec((_NC, _SUB, _LANE), lambda: (0, 0, 0))
    ang_spec = pl.BlockSpec((_SUB, _LANE), lambda: (0, 0))
    per_img_ins = []
    in_specs = []
    for b in range(_B):
        per_img_ins += [scm_t[b], x0t[b], y0t[b], x1t[b], y1t[b], ang_t[b]]
        in_specs += [slab_spec] * 5 + [ang_spec]
    out_specs = []
    out_shape = []
    for b in range(_B):
        out_specs += [pl.BlockSpec((4, _DET), lambda: (0, 0)),
                      pl.BlockSpec((1, _DET), lambda: (0, 0)),
                      pl.BlockSpec((1, _DET), lambda: (0, 0)),
                      pl.BlockSpec((1, _DET), lambda: (0, 0))]
        out_shape += [jax.ShapeDtypeStruct((4, _DET), f32),
                      jax.ShapeDtypeStruct((1, _DET), f32),
                      jax.ShapeDtypeStruct((1, _DET), jnp.int32),
                      jax.ShapeDtypeStruct((1, _DET), f32)]
    scratch_shapes = []
    for b in range(_B):
        scratch_shapes += [pltpu.VMEM((_NC, _SUB, _LANE), f32),
                           pltpu.VMEM((_NC, 1), f32),
                           pltpu.VMEM((_NC, 1), jnp.int32),
                           pltpu.VMEM((_NC, 1), f32),
                           pltpu.VMEM((_NC, 1), f32),
                           pltpu.VMEM((_NC, 1), f32),
                           pltpu.VMEM((_NC, 1), f32),
                           pltpu.VMEM((_NC, 1), f32)]

    nms = pl.pallas_call(
        _nms_kernel,
        grid=(),
        in_specs=in_specs,
        out_specs=out_specs,
        out_shape=out_shape,
        scratch_shapes=scratch_shapes,
    )
    res = nms(*per_img_ins)
    ob = jnp.stack([res[0], res[4]])
    osc = jnp.stack([res[1][0], res[5][0]])
    ol = jnp.stack([res[2][0], res[6][0]])
    oa = jnp.stack([res[3][0], res[7][0]])
    return (jnp.transpose(ob, (0, 2, 1)), osc, ol, oa)


# speedup vs baseline: 25.7482x; 1.0972x over previous
import jax
import jax.numpy as jnp
import numpy as np
from jax.experimental import pallas as pl
from jax.experimental.pallas import tpu as pltpu

_B = 2
_N = 5000
_C = 81
_NC = _C - 1
_DET = 100
_SCORE_THRESH = 0.05
_NMS_THRESH = 0.5
_IMG_H = 800.0
_IMG_W = 800.0
_WX, _WY, _WW, _WH = 10.0, 10.0, 5.0, 5.0
_BBOX_XFORM_CLIP = float(np.log(1000.0 / 16.0))

_TN = 1000
_SUB, _LANE = 8, 625
_BIG = 2**31 - 1
_NEG = -jnp.inf


def _dense_kernel(log_ref, rdx_ref, rdy_ref, rdw_ref, rdh_ref, pr_ref,
                  scm_ref, x0_ref, y0_ref, x1_ref, y1_ref):
    l = log_ref[0]
    m = jnp.max(l, axis=1, keepdims=True)
    e = jnp.exp(l - m)
    s = jnp.sum(e, axis=1, keepdims=True)
    p = e / s
    sc = p[:, 1:]

    pr = pr_ref[0]
    w = pr[:, 2:3] - pr[:, 0:1]
    h = pr[:, 3:4] - pr[:, 1:2]
    cx = pr[:, 0:1] + 0.5 * w
    cy = pr[:, 1:2] + 0.5 * h

    dx = rdx_ref[0][:, 1:] / _WX
    dy = rdy_ref[0][:, 1:] / _WY
    dw = jnp.minimum(rdw_ref[0][:, 1:] / _WW, _BBOX_XFORM_CLIP)
    dh = jnp.minimum(rdh_ref[0][:, 1:] / _WH, _BBOX_XFORM_CLIP)

    pcx = dx * w + cx
    pcy = dy * h + cy
    pw = jnp.exp(dw) * w
    ph = jnp.exp(dh) * h

    x0 = jnp.clip(pcx - 0.5 * pw, 0.0, _IMG_W)
    y0 = jnp.clip(pcy - 0.5 * ph, 0.0, _IMG_H)
    x1 = jnp.clip(pcx + 0.5 * pw, 0.0, _IMG_W)
    y1 = jnp.clip(pcy + 0.5 * ph, 0.0, _IMG_H)

    ws = x1 - x0
    hs = y1 - y0
    valid = (sc > _SCORE_THRESH) & (ws >= 1e-2) & (hs >= 1e-2)
    scm_ref[0] = jnp.where(valid, sc, _NEG)
    x0_ref[0] = x0
    y0_ref[0] = y0
    x1_ref[0] = x1
    y1_ref[0] = y1


def _nms_kernel(*refs):
    ins = [refs[6 * b:6 * b + 6] for b in range(_B)]
    outs = [refs[6 * _B + 4 * b:6 * _B + 4 * b + 4] for b in range(_B)]
    scrs = [refs[10 * _B + 8 * b:10 * _B + 8 * b + 8] for b in range(_B)]

    n2 = (jax.lax.broadcasted_iota(jnp.int32, (_SUB, _LANE), 0) * _LANE
          + jax.lax.broadcasted_iota(jnp.int32, (_SUB, _LANE), 1))
    n3 = (jax.lax.broadcasted_iota(jnp.int32, (_NC, _SUB, _LANE), 1) * _LANE
          + jax.lax.broadcasted_iota(jnp.int32, (_NC, _SUB, _LANE), 2))
    cls80 = jax.lax.broadcasted_iota(jnp.int32, (_NC, 1), 0)
    lane100 = jax.lax.broadcasted_iota(jnp.int32, (1, _DET), 1)

    maxcs = []
    fallbacks = []
    for b in range(_B):
        scm_ref, x0_ref, y0_ref, x1_ref, y1_ref, ang_ref = ins[b]
        s_scr, m_scr, i_scr, cx0_scr, cy0_scr, cx1_scr, cy1_scr, ca_scr = \
            scrs[b]

        scm = scm_ref[...]
        s_scr[...] = scm
        x0a = x0_ref[...]
        y0a = y0_ref[...]
        x1a = x1_ref[...]
        y1a = y1_ref[...]
        anga = ang_ref[...]

        mx = jnp.max(scm, axis=(1, 2), keepdims=True)
        nb = jnp.min(jnp.where(scm == mx, n3, _BIG),
                     axis=(1, 2), keepdims=True)
        best = n3 == nb
        cx0 = jnp.max(jnp.where(best, x0a, -1.0), axis=(1, 2), keepdims=True)
        cy0 = jnp.max(jnp.where(best, y0a, -1.0), axis=(1, 2), keepdims=True)
        cx1 = jnp.max(jnp.where(best, x1a, -1.0), axis=(1, 2), keepdims=True)
        cy1 = jnp.max(jnp.where(best, y1a, -1.0), axis=(1, 2), keepdims=True)
        can = jnp.max(jnp.where(best, anga[None], _NEG),
                      axis=(1, 2), keepdims=True)
        m_scr[...] = mx[:, 0, :]
        i_scr[...] = nb[:, 0, :]
        cx0_scr[...] = cx0[:, 0, :]
        cy0_scr[...] = cy0[:, 0, :]
        cx1_scr[...] = cx1[:, 0, :]
        cy1_scr[...] = cy1[:, 0, :]
        ca_scr[...] = can[:, 0, :]

        maxcs.append(jnp.max(jnp.maximum(jnp.maximum(x0a, y0a),
                                         jnp.maximum(x1a, y1a))))

        s0 = jnp.sum(scm_ref[0, 0:1, 0:1])
        kv0 = s0 > _NEG
        zf = jnp.float32(0.0)
        fallbacks.append((
            jnp.where(kv0, s0, zf),
            jnp.where(kv0, 1, 0),
            jnp.where(kv0, jnp.sum(ang_ref[0:1, 0:1]), zf),
            jnp.where(kv0, jnp.sum(x0_ref[0, 0:1, 0:1]), zf),
            jnp.where(kv0, jnp.sum(y0_ref[0, 0:1, 0:1]), zf),
            jnp.where(kv0, jnp.sum(x1_ref[0, 0:1, 0:1]), zf),
            jnp.where(kv0, jnp.sum(y1_ref[0, 0:1, 0:1]), zf),
        ))

    n2_16 = jnp.concatenate([n2, n2], axis=0)

    def _lane_acc_max(x16):
        a = jnp.maximum(jnp.maximum(x16[:, 0:128], x16[:, 128:256]),
                        jnp.maximum(x16[:, 256:384], x16[:, 384:512]))
        return jnp.maximum(a, x16[:, _LANE - 128:_LANE])

    def _lane_acc_min(x16):
        a = jnp.minimum(jnp.minimum(x16[:, 0:128], x16[:, 128:256]),
                        jnp.minimum(x16[:, 256:384], x16[:, 384:512]))
        return jnp.minimum(a, x16[:, _LANE - 128:_LANE])

    def _split2(r16):
        return (jnp.max(r16[0:8], axis=0, keepdims=True),
                jnp.max(r16[8:16], axis=0, keepdims=True))

    def _split2_min(r16):
        return (jnp.min(r16[0:8], axis=0, keepdims=True),
                jnp.min(r16[8:16], axis=0, keepdims=True))

    def _pair16(a11, b11):
        return jnp.concatenate([jnp.broadcast_to(a11, (8, 1)),
                                jnp.broadcast_to(b11, (8, 1))], axis=0)

    def step(t, carry):
        sel_t = lane100 == t
        sels = []
        for b in range(_B):
            ob_ref, osc_ref, ol_ref, oa_ref = outs[b]
            s_scr, m_scr, i_scr, cx0_scr, cy0_scr, cx1_scr, cy1_scr, \
                ca_scr = scrs[b]

            m = m_scr[...]
            gm11 = jnp.max(m, axis=0, keepdims=True)
            orig = i_scr[...] * _NC + cls80
            kv11 = gm11 > _NEG
            cand = jnp.where((m == gm11) & kv11, orig,
                             jnp.where(kv11, _BIG, 0))
            gidx = jnp.sum(jnp.min(cand, axis=0, keepdims=True))
            gm_s = jnp.sum(gm11)
            kv = gm_s > _NEG
            c = gidx % _NC

            sx0 = jnp.sum(cx0_scr[pl.ds(c, 1), :])
            sy0 = jnp.sum(cy0_scr[pl.ds(c, 1), :])
            sx1 = jnp.sum(cx1_scr[pl.ds(c, 1), :])
            sy1 = jnp.sum(cy1_scr[pl.ds(c, 1), :])
            sang = jnp.sum(ca_scr[pl.ds(c, 1), :])

            fb_s, fb_l, fb_a, fb_x0, fb_y0, fb_x1, fb_y1 = fallbacks[b]
            osc_ref[0:1, :] = jnp.where(sel_t, jnp.where(kv, gm_s, fb_s),
                                        osc_ref[0:1, :])
            ol_ref[0:1, :] = jnp.where(sel_t, jnp.where(kv, c + 1, fb_l),
                                       ol_ref[0:1, :])
            oa_ref[0:1, :] = jnp.where(sel_t, jnp.where(kv, sang, fb_a),
                                       oa_ref[0:1, :])
            ob_ref[0:1, :] = jnp.where(sel_t, jnp.where(kv, sx0, fb_x0),
                                       ob_ref[0:1, :])
            ob_ref[1:2, :] = jnp.where(sel_t, jnp.where(kv, sy0, fb_y0),
                                       ob_ref[1:2, :])
            ob_ref[2:3, :] = jnp.where(sel_t, jnp.where(kv, sx1, fb_x1),
                                       ob_ref[2:3, :])
            ob_ref[3:4, :] = jnp.where(sel_t, jnp.where(kv, sy1, fb_y1),
                                       ob_ref[3:4, :])
            sels.append((c, sx0, sy0, sx1, sy1))

        c0 = sels[0][0]
        c1 = sels[1][0]
        x0_slabs = [ins[b][1][sels[b][0]] for b in range(_B)]
        y0_slabs = [ins[b][2][sels[b][0]] for b in range(_B)]
        x1_slabs = [ins[b][3][sels[b][0]] for b in range(_B)]
        y1_slabs = [ins[b][4][sels[b][0]] for b in range(_B)]

        s_news = []
        for b in range(_B):
            c, sx0, sy0, sx1, sy1 = sels[b]
            off = (c + 1).astype(jnp.float32) * (maxcs[b] + 1.0)
            sx0o, sy0o = sx0 + off, sy0 + off
            sx1o, sy1o = sx1 + off, sy1 + off
            x0o = x0_slabs[b] + off
            y0o = y0_slabs[b] + off
            x1o = x1_slabs[b] + off
            y1o = y1_slabs[b] + off
            ltx = jnp.maximum(sx0o, x0o)
            lty = jnp.maximum(sy0o, y0o)
            rbx = jnp.minimum(sx1o, x1o)
            rby = jnp.minimum(sy1o, y1o)
            iw = jnp.maximum(rbx - ltx, 0.0)
            ih = jnp.maximum(rby - lty, 0.0)
            inter = iw * ih
            a1 = (sx1o - sx0o) * (sy1o - sy0o)
            a2 = (x1o - x0o) * (y1o - y0o)
            iou = inter / (a1 + a2 - inter + 1e-9)
            s_new_b = jnp.where(iou > _NMS_THRESH, _NEG, scrs[b][0][sels[b][0]])
            scrs[b][0][sels[b][0]] = s_new_b
            s_news.append(s_new_b)

        s_new = jnp.concatenate(s_news, axis=0)
        x0_16 = jnp.concatenate(x0_slabs, axis=0)
        y0_16 = jnp.concatenate(y0_slabs, axis=0)
        x1_16 = jnp.concatenate(x1_slabs, axis=0)
        y1_16 = jnp.concatenate(y1_slabs, axis=0)
        ang16 = jnp.concatenate([ins[0][5][...], ins[1][5][...]], axis=0)

        r16 = jnp.max(_lane_acc_max(s_new), axis=1, keepdims=True)
        m2a, m2b = _split2(r16)
        m2s0 = jnp.sum(m2a)
        m2s1 = jnp.sum(m2b)
        eq = jnp.concatenate([s_news[0] == m2s0, s_news[1] == m2s1], axis=0)

        neg1 = jnp.float32(-1.0)
        nbr = jnp.min(_lane_acc_min(jnp.where(eq, n2_16, _BIG)),
                      axis=1, keepdims=True)
        nb2a, nb2b = _split2_min(nbr)
        mxr = jnp.max(_lane_acc_max(jnp.where(eq, n2_16, -1)),
                      axis=1, keepdims=True)
        mxa, mxb = _split2(mxr)
        x0r = jnp.max(_lane_acc_max(jnp.where(eq, x0_16, neg1)),
                      axis=1, keepdims=True)
        nx0a, nx0b = _split2(x0r)
        y0r = jnp.max(_lane_acc_max(jnp.where(eq, y0_16, neg1)),
                      axis=1, keepdims=True)
        ny0a, ny0b = _split2(y0r)
        x1r = jnp.max(_lane_acc_max(jnp.where(eq, x1_16, neg1)),
                      axis=1, keepdims=True)
        nx1a, nx1b = _split2(x1r)
        y1r = jnp.max(_lane_acc_max(jnp.where(eq, y1_16, neg1)),
                      axis=1, keepdims=True)
        ny1a, ny1b = _split2(y1r)
        anr = jnp.max(_lane_acc_max(jnp.where(eq, ang16, _NEG)),
                      axis=1, keepdims=True)
        nana, nanb = _split2(anr)

        per_img = ((c0, m2a, nb2a, mxa, nx0a, ny0a, nx1a, ny1a, nana,
                    s_new[0:8], x0_16[0:8], y0_16[0:8], x1_16[0:8],
                    y1_16[0:8], ang16[0:8]),
                   (c1, m2b, nb2b, mxb, nx0b, ny0b, nx1b, ny1b, nanb,
                    s_new[8:16], x0_16[8:16], y0_16[8:16], x1_16[8:16],
                    y1_16[8:16], ang16[8:16]))
        for b in range(_B):
            (c, m2, nb2, mx_, nx0, ny0, nx1, ny1, nan_,
             s_half, x0h, y0h, x1h, y1h, angh) = per_img[b]
            s_scr, m_scr, i_scr, cx0_scr, cy0_scr, cx1_scr, cy1_scr, \
                ca_scr = scrs[b]
            m_scr[pl.ds(c, 1), :] = m2
            i_scr[pl.ds(c, 1), :] = nb2
            cx0_scr[pl.ds(c, 1), :] = nx0
            cy0_scr[pl.ds(c, 1), :] = ny0
            cx1_scr[pl.ds(c, 1), :] = nx1
            cy1_scr[pl.ds(c, 1), :] = ny1
            ca_scr[pl.ds(c, 1), :] = nan_

            @pl.when(jnp.sum(mx_) != jnp.sum(nb2))
            def _tie_fixup(c=c, nb2=nb2, s_half=s_half, x0h=x0h, y0h=y0h,
                           x1h=x1h, y1h=y1h, angh=angh, cx0_scr=cx0_scr,
                           cy0_scr=cy0_scr, cx1_scr=cx1_scr, cy1_scr=cy1_scr,
                           ca_scr=ca_scr):
                best2 = n2 == nb2
                cx0_scr[pl.ds(c, 1), :] = jnp.max(
                    jnp.where(best2, x0h, neg1), axis=(0, 1), keepdims=True)
                cy0_scr[pl.ds(c, 1), :] = jnp.max(
                    jnp.where(best2, y0h, neg1), axis=(0, 1), keepdims=True)
                cx1_scr[pl.ds(c, 1), :] = jnp.max(
                    jnp.where(best2, x1h, neg1), axis=(0, 1), keepdims=True)
                cy1_scr[pl.ds(c, 1), :] = jnp.max(
                    jnp.where(best2, y1h, neg1), axis=(0, 1), keepdims=True)
                ca_scr[pl.ds(c, 1), :] = jnp.max(
                    jnp.where(best2, angh, _NEG), axis=(0, 1), keepdims=True)
        return carry

    jax.lax.fori_loop(0, _DET, step, jnp.int32(0))


def _cm(a):
    return jnp.transpose(a, (0, 2, 1)).reshape(_B, _NC, _SUB, _LANE)


@jax.jit
def kernel(class_logits, box_regression, angle_pred, proposals):
    r = box_regression.reshape(_B, _N, _C, 4)
    rdx = r[..., 0]
    rdy = r[..., 1]
    rdw = r[..., 2]
    rdh = r[..., 3]

    ntiles = _N // _TN
    f32 = jnp.float32
    dense = pl.pallas_call(
        _dense_kernel,
        grid=(_B, ntiles),
        in_specs=[
            pl.BlockSpec((1, _TN, _C), lambda b, t: (b, t, 0)),
            pl.BlockSpec((1, _TN, _C), lambda b, t: (b, t, 0)),
            pl.BlockSpec((1, _TN, _C), lambda b, t: (b, t, 0)),
            pl.BlockSpec((1, _TN, _C), lambda b, t: (b, t, 0)),
            pl.BlockSpec((1, _TN, _C), lambda b, t: (b, t, 0)),
            pl.BlockSpec((1, _TN, 4), lambda b, t: (b, t, 0)),
        ],
        out_specs=[
            pl.BlockSpec((1, _TN, _NC), lambda b, t: (b, t, 0))
            for _ in range(5)
        ],
        out_shape=[jax.ShapeDtypeStruct((_B, _N, _NC), f32) for _ in range(5)],
        compiler_params=pltpu.CompilerParams(
            dimension_semantics=("parallel", "arbitrary")),
    )
    scm, x0, y0, x1, y1 = dense(class_logits, rdx, rdy, rdw, rdh, proposals)

    scm_t = _cm(scm)
    x0t = _cm(x0)
    y0t = _cm(y0)
    x1t = _cm(x1)
    y1t = _cm(y1)
    ang_t = jnp.transpose(angle_pred, (0, 2, 1)).reshape(_B, _SUB, _LANE)

    slab_spec = pl.BlockSpec((_NC, _SUB, _LANE), lambda: (0, 0, 0))
    ang_spec = pl.BlockSpec((_SUB, _LANE), lambda: (0, 0))
    per_img_ins = []
    in_specs = []
    for b in range(_B):
        per_img_ins += [scm_t[b], x0t[b], y0t[b], x1t[b], y1t[b], ang_t[b]]
        in_specs += [slab_spec] * 5 + [ang_spec]
    out_specs = []
    out_shape = []
    for b in range(_B):
        out_specs += [pl.BlockSpec((4, _DET), lambda: (0, 0)),
                      pl.BlockSpec((1, _DET), lambda: (0, 0)),
                      pl.BlockSpec((1, _DET), lambda: (0, 0)),
                      pl.BlockSpec((1, _DET), lambda: (0, 0))]
        out_shape += [jax.ShapeDtypeStruct((4, _DET), f32),
                      jax.ShapeDtypeStruct((1, _DET), f32),
                      jax.ShapeDtypeStruct((1, _DET), jnp.int32),
                      jax.ShapeDtypeStruct((1, _DET), f32)]
    scratch_shapes = []
    for b in range(_B):
        scratch_shapes += [pltpu.VMEM((_NC, _SUB, _LANE), f32),
                           pltpu.VMEM((_NC, 1), f32),
                           pltpu.VMEM((_NC, 1), jnp.int32),
                           pltpu.VMEM((_NC, 1), f32),
                           pltpu.VMEM((_NC, 1), f32),
                           pltpu.VMEM((_NC, 1), f32),
                           pltpu.VMEM((_NC, 1), f32),
                           pltpu.VMEM((_NC, 1), f32)]

    nms = pl.pallas_call(
        _nms_kernel,
        grid=(),
        in_specs=in_specs,
        out_specs=out_specs,
        out_shape=out_shape,
        scratch_shapes=scratch_shapes,
    )
    res = nms(*per_img_ins)
    ob = jnp.stack([res[0], res[4]])
    osc = jnp.stack([res[1][0], res[5][0]])
    ol = jnp.stack([res[2][0], res[6][0]])
    oa = jnp.stack([res[3][0], res[7][0]])
    return (jnp.transpose(ob, (0, 2, 1)), osc, ol, oa)
